# Initial kernel scaffold; baseline (speedup 1.0000x reference)
import jax
import jax.numpy as jnp
from jax.experimental import pallas as pl


def kernel(text_idxs, text_len, embedding_table):
    raise NotImplementedError("write your pallas kernel here")


# speedup vs baseline: 3.9147x; 3.9147x over previous
import functools

import jax
import jax.numpy as jnp
from jax import lax
from jax.experimental import pallas as pl
from jax.experimental.pallas import tpu as pltpu
from jax.experimental.pallas import tpu_sc as plsc

B = 4096
L = 50
D = 64
LANES = 16
NSEG = D // LANES

NC, NS = 2, 16
NW = NC * NS
BPW = B // NW
CH = 2
NCHUNK = BPW // CH
IDX_RAW = CH * L
IDX_PAD = 104


_GATHER_DNUMS = lax.GatherDimensionNumbers(
    offset_dims=(), collapsed_slice_dims=(0,), start_index_map=(0,)
)


def _lane_shuffle(v, idx):
    return lax.gather(
        v,
        idx[:, None],
        dimension_numbers=_GATHER_DNUMS,
        slice_sizes=(1,),
        mode=lax.GatherScatterMode.PROMISE_IN_BOUNDS,
    )


def _allsum16(v):
    lane = lax.iota(jnp.int32, LANES)
    for s in (1, 2, 4, 8):
        v = v + _lane_shuffle(v, jnp.bitwise_xor(lane, s))
    return v


def _sc_body(idx_hbm, table_hbm, out_hbm, idx_v, buf, acc_v, sem):
    wid = lax.axis_index("s") * NC + lax.axis_index("c")

    pltpu.sync_copy(idx_hbm.at[wid], idx_v)

    def chunk_body(j, carry):
        pltpu.async_copy(table_hbm.at[idx_v.at[j]], buf, sem).wait()
        for c in range(CH):
            base = c * L
            segs = [buf[base, pl.ds(k * LANES, LANES)] for k in range(NSEG)]
            for l in range(1, L):
                for k in range(NSEG):
                    segs[k] = segs[k] + buf[base + l, pl.ds(k * LANES, LANES)]
            r = j * CH + c
            for k in range(NSEG):
                acc_v[r, pl.ds(k * LANES, LANES)] = segs[k]
        return carry

    lax.fori_loop(0, NCHUNK, chunk_body, 0, unroll=False)

    def norm_body(r, carry):
        segs = [acc_v[r, pl.ds(k * LANES, LANES)] for k in range(NSEG)]
        v = segs[0] * segs[0]
        for k in range(1, NSEG):
            v = v + segs[k] * segs[k]
        sv = jnp.maximum(_allsum16(v), jnp.float32(1e-24))
        yi = jnp.full((LANES,), 0x5F3759DF, dtype=jnp.int32) - (
            lax.shift_right_logical(lax.bitcast_convert_type(sv, jnp.int32), 1)
        )
        y = lax.bitcast_convert_type(yi, jnp.float32)
        half = sv * jnp.float32(0.5)
        for _ in range(3):
            y = y * (jnp.float32(1.5) - half * y * y)
        for k in range(NSEG):
            acc_v[r, pl.ds(k * LANES, LANES)] = segs[k] * y
        return carry

    lax.fori_loop(0, BPW, norm_body, 0, unroll=False)

    pltpu.sync_copy(acc_v, out_hbm.at[pl.ds(wid * BPW, BPW)])


@jax.jit
def _mean_embed(idx_blocks, table):
    mesh = plsc.VectorSubcoreMesh(core_axis_name="c", subcore_axis_name="s")
    f = pl.kernel(
        _sc_body,
        out_type=jax.ShapeDtypeStruct((B, D), jnp.float32),
        mesh=mesh,
        compiler_params=pltpu.CompilerParams(use_tc_tiling_on_sc=False),
        scratch_types=[
            pltpu.VMEM((NCHUNK, IDX_PAD), jnp.int32),
            pltpu.VMEM((IDX_PAD, D), jnp.float32),
            pltpu.VMEM((BPW, D), jnp.float32),
            pltpu.SemaphoreType.DMA,
        ],
    )
    return f(idx_blocks, table)


def kernel(text_idxs, text_len, embedding_table):
    del text_len
    idx = text_idxs.astype(jnp.int32).reshape(NW, NCHUNK, IDX_RAW)
    idx = jnp.pad(idx, ((0, 0), (0, 0), (0, IDX_PAD - IDX_RAW)))
    return _mean_embed(idx, embedding_table)
